# baseline (speedup 1.0000x reference)
import functools

import jax
import jax.numpy as jnp
from jax import lax
from jax.experimental import pallas as pl
from jax.experimental.pallas import tpu as pltpu
from jax.experimental.pallas import tpu_sc as plsc

_W = 4
_K = 12
_TW = 253
_T = 256
_IMIN = -2147483648
_NEG = float("-inf")

_NBC = 1024
_NBC_SC = 288
_NW = 32
_ROWS_SC = _NBC_SC * 128
_RPW = _ROWS_SC // _NW
_RC = 128
_NCH = _RPW // _RC

_R = 1024
_TILES0 = _ROWS_SC // _R


def _tc_body(x_ref, o_ref):
    xv = x_ref[...].reshape(_R, _T)
    w = xv
    w += jnp.concatenate([xv[:, 1:], xv[:, :1]], axis=1)
    w += jnp.concatenate([xv[:, 2:], xv[:, :2]], axis=1)
    w += jnp.concatenate([xv[:, 3:], xv[:, :3]], axis=1)
    bits = jax.lax.bitcast_convert_type(w, jnp.int32)
    key = jnp.where(bits < 0, bits ^ 0x7FFFFFFF, bits)
    t = jax.lax.broadcasted_iota(jnp.int32, (_R, _T), 1)
    key = (key & -256) | t
    key = jnp.where(t < _TW, key, _IMIN)
    kt = key.T

    v = kt.reshape(16, 2, 8, _R)
    a, b = v[:, 0], v[:, 1]
    p = jnp.maximum(a, b)
    mn = jnp.minimum(a, b)
    acc = jnp.zeros((1, 1, _R), jnp.float32)
    for _ in range(_K):
        m = jnp.max(p, axis=(0, 1), keepdims=True)
        kv = m & -256
        vbits = jnp.where(kv < 0, kv ^ 0x7FFFFFFF, kv)
        acc += jax.lax.bitcast_convert_type(vbits, jnp.float32)
        eq = p == m
        p = jnp.where(eq, mn, p)
        mn = jnp.where(eq, _IMIN, mn)
    zz = jnp.mean(acc.reshape(_R // 128, 128), axis=1) * (1.0 / (_K * _W))
    o_ref[0] = jnp.broadcast_to(zz[:, None], (_R // 128, 128))


def _tc_call(x):
    ntc = _NBC - _NBC_SC
    xr = x.reshape(_NBC * 128 // _R, _R, _T)
    grid = ntc * 128 // _R
    out = pl.pallas_call(
        _tc_body,
        grid=(grid,),
        in_specs=[pl.BlockSpec((1, _R, _T), lambda i: (i + _TILES0, 0, 0))],
        out_specs=pl.BlockSpec((1, _R // 128, 128), lambda i: (i, 0, 0)),
        out_shape=jax.ShapeDtypeStruct((grid, _R // 128, 128), jnp.float32),
    )(xr)
    return out[:, :, 0].reshape(ntc)


def _sc_insert(s, w):
    out = []
    for i in range(_K):
        hi = jnp.maximum(s[i], w)
        w = jnp.minimum(s[i], w)
        out.append(hi)
    return out


def _sc_body(x_hbm, o_hbm, buf, obuf):
    wid = lax.axis_index("s") * 2 + lax.axis_index("c")

    def chunk(ci, _):
        base = wid * _RPW + ci * _RC
        pltpu.sync_copy(x_hbm.at[pl.ds(base * _T, _RC * _T)], buf)
        psum = jnp.zeros((16,), jnp.float32)
        for gp in range(_RC // 32):
            rowv = [(g * 16 + lax.iota(jnp.int32, 16)) * _T
                    for g in (2 * gp, 2 * gp + 1)]
            a = [[plsc.load_gather(buf, [rv + t]) for t in range(4)]
                 for rv in rowv]
            w = [aa[0] + aa[1] + aa[2] + aa[3] for aa in a]
            s = [[jnp.full((16,), _NEG, jnp.float32) for _ in range(_K)]
                 for _ in range(2)]
            s = [_sc_insert(s[q], w[q]) for q in range(2)]

            def tstep(j, carry):
                w0, w1, tv, a0, a1, s0, s1 = carry
                w, aa, ss = [w0, w1], [list(a0), list(a1)], [list(s0), list(s1)]
                for u in range(4):
                    for q in range(2):
                        xn = plsc.load_gather(buf, [rowv[q] + tv])
                        w[q] = w[q] + xn - aa[q][u]
                        ss[q] = _sc_insert(ss[q], w[q])
                        aa[q][u] = xn
                    tv = tv + 1
                return (w[0], w[1], tv,
                        tuple(aa[0]), tuple(aa[1]), tuple(ss[0]), tuple(ss[1]))

            tv0 = jnp.full((16,), 4, jnp.int32)
            carry = (w[0], w[1], tv0,
                     tuple(a[0]), tuple(a[1]), tuple(s[0]), tuple(s[1]))
            carry = lax.fori_loop(0, 63, tstep, carry)
            for q in range(2):
                sq = list(carry[5 + q])
                z = sq[0]
                for i in range(1, _K):
                    z = z + sq[i]
                psum = psum + z
        obuf[pl.ds(pl.multiple_of(ci * 16, 8), 16)] = psum
        return 0

    lax.fori_loop(0, _NCH, chunk, 0)
    pltpu.sync_copy(obuf, o_hbm.at[pl.ds(wid * _NCH * 16, _NCH * 16)])


def _sc_call(x):
    xs = x.reshape(_NBC * 128, _T)[:_ROWS_SC]
    x2 = xs.reshape(_ROWS_SC * _T)
    mesh = plsc.VectorSubcoreMesh(core_axis_name="c", subcore_axis_name="s")
    k = functools.partial(
        pl.kernel,
        out_type=jax.ShapeDtypeStruct((_NBC_SC * 16,), jnp.float32),
        mesh=mesh,
        scratch_types=[
            pltpu.VMEM((_RC * _T,), jnp.float32),
            pltpu.VMEM((_NCH * 16,), jnp.float32),
        ],
        compiler_params=pltpu.CompilerParams(
            use_tc_tiling_on_sc=False, needs_layout_passes=False),
    )(_sc_body)
    out = k(x2)
    return jnp.sum(out.reshape(_NBC_SC, 16), axis=1) * (1.0 / (128 * _K * _W))


@jax.jit
def kernel(x):
    B, C, F, T = x.shape
    z_sc = _sc_call(x)
    z_tc = _tc_call(x)
    return jnp.concatenate([z_sc, z_tc]).reshape(B, C)


# speedup vs baseline: 1.1779x; 1.1779x over previous
import functools

import jax
import jax.numpy as jnp
from jax import lax
from jax.experimental import pallas as pl
from jax.experimental.pallas import tpu as pltpu
from jax.experimental.pallas import tpu_sc as plsc

_W = 4
_K = 12
_TW = 253
_T = 256
_IMIN = -2147483648
_NEG = float("-inf")

_NBC = 1024
_NBC_SC = 288
_NW = 32
_ROWS_SC = _NBC_SC * 128
_RPW = _ROWS_SC // _NW
_RC = 128
_NCH = _RPW // _RC

_R = 1024
_TILES0 = _ROWS_SC // _R


def _tc_body(x_ref, o_ref):
    xv = x_ref[...].reshape(_R, _T)
    w = xv
    w += jnp.concatenate([xv[:, 1:], xv[:, :1]], axis=1)
    w += jnp.concatenate([xv[:, 2:], xv[:, :2]], axis=1)
    w += jnp.concatenate([xv[:, 3:], xv[:, :3]], axis=1)
    bits = jax.lax.bitcast_convert_type(w, jnp.int32)
    key = jnp.where(bits < 0, bits ^ 0x7FFFFFFF, bits)
    t = jax.lax.broadcasted_iota(jnp.int32, (_R, _T), 1)
    key = (key & -256) | t
    key = jnp.where(t < _TW, key, _IMIN)
    kt = key.T

    v = kt.reshape(16, 2, 8, _R)
    a, b = v[:, 0], v[:, 1]
    p = jnp.maximum(a, b)
    mn = jnp.minimum(a, b)
    acc = jnp.zeros((1, 1, _R), jnp.float32)
    for _ in range(_K):
        m = jnp.max(p, axis=(0, 1), keepdims=True)
        kv = m & -256
        vbits = jnp.where(kv < 0, kv ^ 0x7FFFFFFF, kv)
        acc += jax.lax.bitcast_convert_type(vbits, jnp.float32)
        eq = p == m
        p = jnp.where(eq, mn, p)
        mn = jnp.where(eq, _IMIN, mn)
    zz = jnp.mean(acc.reshape(_R // 128, 128), axis=1) * (1.0 / (_K * _W))
    o_ref[0] = jnp.broadcast_to(zz[:, None], (_R // 128, 128))


def _tc_call(x):
    ntc = _NBC - _NBC_SC
    xr = x.reshape(_NBC * 128 // _R, _R, _T)
    grid = ntc * 128 // _R
    out = pl.pallas_call(
        _tc_body,
        grid=(grid,),
        in_specs=[pl.BlockSpec((1, _R, _T), lambda i: (i + _TILES0, 0, 0))],
        out_specs=pl.BlockSpec((1, _R // 128, 128), lambda i: (i, 0, 0)),
        out_shape=jax.ShapeDtypeStruct((grid, _R // 128, 128), jnp.float32),
    )(xr)
    return out[:, :, 0].reshape(ntc)


def _sc_insert(s, w):
    out = []
    for i in range(_K):
        hi = jnp.maximum(s[i], w)
        w = jnp.minimum(s[i], w)
        out.append(hi)
    return out


def _sc_body(x_hbm, o_hbm, buf, obuf):
    wid = lax.axis_index("s") * 2 + lax.axis_index("c")

    def chunk(ci, _):
        base = wid * _RPW + ci * _RC
        pltpu.sync_copy(x_hbm.at[pl.ds(base * _T, _RC * _T)], buf)
        psum = jnp.zeros((16,), jnp.float32)
        for g in range(_RC // 16):
            rowv = (g * 16 + lax.iota(jnp.int32, 16)) * _T
            a = [plsc.load_gather(buf, [rowv + t]) for t in range(4)]
            w = a[0] + a[1] + a[2] + a[3]
            s = [jnp.full((16,), _NEG, jnp.float32) for _ in range(_K)]
            s = _sc_insert(s, w)

            def tstep(j, carry):
                w, a0, a1, a2, a3, tv, *s = carry
                aa = [a0, a1, a2, a3]
                nw = []
                for u in range(4):
                    xn = plsc.load_gather(buf, [rowv + tv])
                    tv = tv + 1
                    w = w + xn - aa[u]
                    s = _sc_insert(s, w)
                    nw.append(xn)
                return (w, nw[0], nw[1], nw[2], nw[3], tv, *s)

            tv0 = jnp.full((16,), 4, jnp.int32)
            carry = (w, a[0], a[1], a[2], a[3], tv0, *s)
            carry = lax.fori_loop(0, 63, tstep, carry)
            s = list(carry[6:])
            z = s[0]
            for i in range(1, _K):
                z = z + s[i]
            psum = psum + z
        obuf[pl.ds(pl.multiple_of(ci * 16, 8), 16)] = psum
        return 0

    lax.fori_loop(0, _NCH, chunk, 0)
    pltpu.sync_copy(obuf, o_hbm.at[pl.ds(wid * _NCH * 16, _NCH * 16)])


def _sc_call(x):
    xs = x.reshape(_NBC * 128, _T)[:_ROWS_SC]
    x2 = xs.reshape(_ROWS_SC * _T)
    mesh = plsc.VectorSubcoreMesh(core_axis_name="c", subcore_axis_name="s")
    k = functools.partial(
        pl.kernel,
        out_type=jax.ShapeDtypeStruct((_NBC_SC * 16,), jnp.float32),
        mesh=mesh,
        scratch_types=[
            pltpu.VMEM((_RC * _T,), jnp.float32),
            pltpu.VMEM((_NCH * 16,), jnp.float32),
        ],
        compiler_params=pltpu.CompilerParams(
            use_tc_tiling_on_sc=False, needs_layout_passes=False),
    )(_sc_body)
    out = k(x2)
    return jnp.sum(out.reshape(_NBC_SC, 16), axis=1) * (1.0 / (128 * _K * _W))


@jax.jit
def kernel(x):
    B, C, F, T = x.shape
    z_sc = _sc_call(x)
    z_tc = _tc_call(x)
    return jnp.concatenate([z_sc, z_tc]).reshape(B, C)
